# baseline (speedup 1.0000x reference)
import functools

import jax
import jax.numpy as jnp
from jax import lax
from jax.experimental import pallas as pl
from jax.experimental.pallas import tpu as pltpu
from jax.experimental.pallas import tpu_sc as plsc

D = 64
NC, NS = 2, 16
NW = NC * NS
CH = 128
S = 50


@jax.jit
def _gather_rows(idx, table):
    mesh = plsc.VectorSubcoreMesh(core_axis_name="c", subcore_axis_name="s")

    @functools.partial(
        pl.kernel,
        out_type=jax.ShapeDtypeStruct((S, 8, NW, 8, CH), jnp.float32),
        mesh=mesh,
        scratch_types=[
            pltpu.VMEM((S, CH), jnp.int32),
            pltpu.VMEM((2, CH, D), jnp.float32),
            pltpu.VMEM((2, 8, 8, CH), jnp.float32),
            pltpu.SemaphoreType.DMA,
            pltpu.SemaphoreType.DMA,
            pltpu.SemaphoreType.DMA,
            pltpu.SemaphoreType.DMA,
        ],
        compiler_params=pltpu.CompilerParams(
            use_tc_tiling_on_sc=False, needs_layout_passes=False
        ),
    )
    def k(idx_hbm, table_hbm, out_hbm, idx_v, gbuf, tbuf, gs0, gs1, ws0, ws1):
        wid = lax.axis_index("s") * NC + lax.axis_index("c")
        pltpu.sync_copy(idx_hbm.at[:, wid], idx_v)
        iota = lax.iota(jnp.int32, 16)
        gsems, wsems = (gs0, gs1), (ws0, ws1)

        def gfire(sq, p):
            pltpu.async_copy(table_hbm.at[idx_v.at[sq]], gbuf.at[p], gsems[p])

        def gwait(sq, p):
            pltpu.make_async_copy(
                table_hbm.at[idx_v.at[sq]], gbuf.at[p], gsems[p]
            ).wait()

        def wfire(sq, p):
            pltpu.async_copy(tbuf.at[p], out_hbm.at[sq, :, wid], wsems[p])

        def wwait(sq, p):
            pltpu.make_async_copy(
                tbuf.at[p], out_hbm.at[sq, :, wid], wsems[p]
            ).wait()

        def transpose(p):
            for d in range(D):
                dcol = jnp.full((16,), d, jnp.int32)
                for j in range(8):
                    v = plsc.load_gather(gbuf.at[p], [iota + 16 * j, dcol])
                    tbuf[p, d // 8, d % 8, pl.ds(16 * j, 16)] = v

        gfire(0, 0)

        def body(i, carry):
            s = 2 * i
            gwait(s, 0)
            gfire(s + 1, 1)

            @pl.when(i > 0)
            def _w0():
                wwait(s - 2, 0)

            transpose(0)
            wfire(s, 0)

            gwait(s + 1, 1)

            @pl.when(s + 2 < S)
            def _g0():
                gfire(s + 2, 0)

            @pl.when(i > 0)
            def _w1():
                wwait(s - 1, 1)

            transpose(1)
            wfire(s + 1, 1)
            return carry

        lax.fori_loop(0, S // 2, body, 0, unroll=False)
        wwait(S - 2, 0)
        wwait(S - 1, 1)

    return k(idx, table)


def kernel(x, emb_weight):
    idx = x.T.reshape(S, NW, CH)
    out5 = _gather_rows(idx, emb_weight)
    out = out5.transpose(0, 1, 3, 2, 4).reshape(S, D, NW * CH).transpose(2, 0, 1)
    return out


# speedup vs baseline: 2.0226x; 2.0226x over previous
import functools

import jax
import jax.numpy as jnp
from jax import lax
from jax.experimental import pallas as pl
from jax.experimental.pallas import tpu as pltpu
from jax.experimental.pallas import tpu_sc as plsc

D = 64
NC, NS = 2, 16
NW = NC * NS
CH = 128
S = 50


@jax.jit
def _gather_rows(idx, table):
    mesh = plsc.VectorSubcoreMesh(core_axis_name="c", subcore_axis_name="s")

    @functools.partial(
        pl.kernel,
        out_type=jax.ShapeDtypeStruct((S, 8, NW, 8, CH), jnp.float32),
        mesh=mesh,
        scratch_types=[
            pltpu.VMEM((S, CH), jnp.int32),
            pltpu.VMEM((2, CH, D), jnp.float32),
            pltpu.VMEM((2, D, 129), jnp.float32),
            pltpu.SemaphoreType.DMA,
            pltpu.SemaphoreType.DMA,
            pltpu.SemaphoreType.DMA,
            pltpu.SemaphoreType.DMA,
        ],
        compiler_params=pltpu.CompilerParams(
            use_tc_tiling_on_sc=False, needs_layout_passes=False
        ),
    )
    def k(idx_hbm, table_hbm, out_hbm, idx_v, gbuf, tbuf, gs0, gs1, ws0, ws1):
        wid = lax.axis_index("s") * NC + lax.axis_index("c")
        pltpu.sync_copy(idx_hbm.at[:, wid], idx_v)
        iota = lax.iota(jnp.int32, 16)
        gsems, wsems = (gs0, gs1), (ws0, ws1)

        def gfire(sq, p):
            pltpu.async_copy(table_hbm.at[idx_v.at[sq]], gbuf.at[p], gsems[p])

        def gwait(sq, p):
            pltpu.make_async_copy(
                table_hbm.at[idx_v.at[sq]], gbuf.at[p], gsems[p]
            ).wait()

        def wfire(sq, p):
            for tr in range(8):
                pltpu.async_copy(
                    tbuf.at[p, pl.ds(8 * tr, 8), pl.ds(0, CH)],
                    out_hbm.at[sq, tr, wid],
                    wsems[p],
                )

        def wwait(sq, p):
            for tr in range(8):
                pltpu.make_async_copy(
                    tbuf.at[p, pl.ds(8 * tr, 8), pl.ds(0, CH)],
                    out_hbm.at[sq, tr, wid],
                    wsems[p],
                ).wait()

        djs = [iota + 16 * j for j in range(D // 16)]

        def transpose(p):
            for c in range(CH):
                cv = jnp.full((16,), c, jnp.int32)
                for j in range(D // 16):
                    v = gbuf[p, c, pl.ds(16 * j, 16)]
                    plsc.store_scatter(tbuf.at[p], [djs[j], cv], v)

        gfire(0, 0)

        def body(i, carry):
            s = 2 * i
            gwait(s, 0)
            gfire(s + 1, 1)

            @pl.when(i > 0)
            def _w0():
                wwait(s - 2, 0)

            transpose(0)
            wfire(s, 0)

            gwait(s + 1, 1)

            @pl.when(s + 2 < S)
            def _g0():
                gfire(s + 2, 0)

            @pl.when(i > 0)
            def _w1():
                wwait(s - 1, 1)

            transpose(1)
            wfire(s + 1, 1)
            return carry

        lax.fori_loop(0, S // 2, body, 0, unroll=False)
        wwait(S - 2, 0)
        wwait(S - 1, 1)

    return k(idx, table)


def kernel(x, emb_weight):
    idx = x.T.reshape(S, NW, CH)
    out5 = _gather_rows(idx, emb_weight)
    out = out5.transpose(0, 1, 3, 2, 4).reshape(S, D, NW * CH).transpose(2, 0, 1)
    return out


# speedup vs baseline: 2.0238x; 1.0006x over previous
import functools

import jax
import jax.numpy as jnp
from jax import lax
from jax.experimental import pallas as pl
from jax.experimental.pallas import tpu as pltpu
from jax.experimental.pallas import tpu_sc as plsc

D = 64
NC, NS = 2, 16
NW = NC * NS
CH = 128
S = 50


@jax.jit
def _gather_rows(idx, table):
    mesh = plsc.VectorSubcoreMesh(core_axis_name="c", subcore_axis_name="s")

    @functools.partial(
        pl.kernel,
        out_type=jax.ShapeDtypeStruct((S, 8, NW, 8, CH), jnp.float32),
        mesh=mesh,
        scratch_types=[
            pltpu.VMEM((S, CH), jnp.int32),
            pltpu.VMEM((2, CH, D), jnp.float32),
            pltpu.VMEM((2, D, 129), jnp.float32),
            pltpu.SemaphoreType.DMA,
            pltpu.SemaphoreType.DMA,
            pltpu.SemaphoreType.DMA,
            pltpu.SemaphoreType.DMA,
        ],
        compiler_params=pltpu.CompilerParams(
            use_tc_tiling_on_sc=False, needs_layout_passes=False
        ),
    )
    def k(idx_hbm, table_hbm, out_hbm, idx_v, gbuf, tbuf, gs0, gs1, ws0, ws1):
        wid = lax.axis_index("s") * NC + lax.axis_index("c")
        pltpu.sync_copy(idx_hbm.at[:, wid], idx_v)
        iota = lax.iota(jnp.int32, 16)
        gsems, wsems = (gs0, gs1), (ws0, ws1)

        def gfire(sq, p):
            pltpu.async_copy(table_hbm.at[idx_v.at[sq]], gbuf.at[p], gsems[p])

        def gwait(sq, p):
            pltpu.make_async_copy(
                table_hbm.at[idx_v.at[sq]], gbuf.at[p], gsems[p]
            ).wait()

        def wfire(sq, p):
            for tr in range(8):
                pltpu.async_copy(
                    tbuf.at[p, pl.ds(8 * tr, 8), pl.ds(0, CH)],
                    out_hbm.at[sq, tr, wid],
                    wsems[p],
                )

        def wwait(sq, p):
            for tr in range(8):
                pltpu.make_async_copy(
                    tbuf.at[p, pl.ds(8 * tr, 8), pl.ds(0, CH)],
                    out_hbm.at[sq, tr, wid],
                    wsems[p],
                ).wait()

        djs = [iota + 16 * j for j in range(D // 16)]

        ones = jnp.full((16,), 1, jnp.int32)

        def transpose(p):
            cv = jnp.full((16,), 0, jnp.int32)
            for c in range(CH):
                for j in range(D // 16):
                    v = gbuf[p, c, pl.ds(16 * j, 16)]
                    plsc.store_scatter(tbuf.at[p], [djs[j], cv], v)
                cv = cv + ones

        gfire(0, 0)

        def body(i, carry):
            s = 2 * i
            gwait(s, 0)
            gfire(s + 1, 1)

            @pl.when(i > 0)
            def _w0():
                wwait(s - 2, 0)

            transpose(0)
            wfire(s, 0)

            gwait(s + 1, 1)

            @pl.when(s + 2 < S)
            def _g0():
                gfire(s + 2, 0)

            @pl.when(i > 0)
            def _w1():
                wwait(s - 1, 1)

            transpose(1)
            wfire(s + 1, 1)
            return carry

        lax.fori_loop(0, S // 2, body, 0, unroll=False)
        wwait(S - 2, 0)
        wwait(S - 1, 1)

    return k(idx, table)


def kernel(x, emb_weight):
    idx = x.T.reshape(S, NW, CH)
    out5 = _gather_rows(idx, emb_weight)
    out = out5.transpose(0, 1, 3, 2, 4).reshape(S, D, NW * CH).transpose(2, 0, 1)
    return out
